# baseline (speedup 1.0000x reference)
import functools
import math

import jax
import jax.numpy as jnp
from jax import lax
from jax.experimental import pallas as pl
from jax.experimental.pallas import tpu as pltpu
from jax.experimental.pallas import tpu_sc as plsc

N_BINS = 100
BW = 1.5 / N_BINS
INV_BW = N_BINS / 1.5
C1 = math.exp(-2.0)
C2 = math.exp(-8.0)

EV = [1.0000000005495513, 0.4999999725452058, 0.04166688603182874,
      0.001388275939515883, 2.5499197512124858e-05]
OD = [1.0000000002742597, 0.1666666611856373, 0.008333363992309727,
      0.00019834275296320354, 2.8254136118362913e-06]
PB = [0.9999999998500073, -0.4999999852294344, 0.12499976213798546,
      -0.020831893152229372, 0.0025999953457805794, -0.0002541579152189521,
      1.693882112112576e-05]

V = 64 * 64 * 64
NC = 2
NS = 16
L = 16
NW = NC * NS
VPW = V // NS
UNROLL = 4
ITERS = VPW // (L * UNROLL)
HSTRIDE = 129
HWORDS = L * HSTRIDE
OUTW = 384


def _sc_body(p_hbm, t_hbm, c_hbm, out_hbm, dp, dt, sd, histp, histt, orow,
             semp, semt, sems):
    c = lax.axis_index("c")
    s = lax.axis_index("s")
    wid = c * NS + s
    zoff = s * 4

    cp_p = pltpu.async_copy(p_hbm.at[c, 0, pl.ds(zoff, 4)], dp, semp)
    cp_t = pltpu.async_copy(t_hbm.at[c, 0, pl.ds(zoff, 4)], dt, semt)
    cp_s = pltpu.async_copy(c_hbm.at[c, 1, pl.ds(zoff, 4)], sd, sems)

    zeros = jnp.zeros((L,), jnp.float32)

    def zero_body(j, _):
        histp[pl.ds(j * L, L)] = zeros
        histt[pl.ds(j * L, L)] = zeros
        return 0

    lax.fori_loop(0, HSTRIDE, zero_body, 0)

    def zero_orow(j, _):
        orow[pl.ds(j * L, L)] = zeros
        return 0

    lax.fori_loop(0, OUTW // L, zero_orow, 0)

    cp_p.wait()
    cp_t.wait()
    cp_s.wait()

    lane_base = lax.iota(jnp.int32, L) * HSTRIDE

    def process(d, mf, href):
        t = d * INV_BW
        k = t.astype(jnp.int32)
        k = jnp.minimum(k, 96)
        f1 = (t - k.astype(jnp.float32)) * 2.0 - 1.0
        base = jnp.exp(-0.5 * (f1 * f1))
        g = jnp.exp(f1 + f1)
        gi = 1.0 / g
        w1 = base * (g * C1)
        w2 = base * ((g * g) * C2)
        v1 = base * (gi * C1)
        v2 = base * ((gi * gi) * C2)
        m1 = k >= 1
        m2 = k >= 2
        z = jnp.float32(0.0)
        ssum = ((base + w1) + w2) + (jnp.where(m1, v1, z) + jnp.where(m2, v2, z))
        scale = mf / (ssum + 1e-8)
        bidx = lane_base + k
        plsc.addupdate_scatter(href, [bidx], base * scale)
        plsc.addupdate_scatter(href, [bidx + 1], w1 * scale)
        plsc.addupdate_scatter(href, [bidx + 2], w2 * scale)
        plsc.addupdate_scatter(href, [bidx - 1], v1 * scale, mask=m1)
        plsc.addupdate_scatter(href, [bidx - 2], v2 * scale, mask=m2)

    @plsc.parallel_loop(0, 256, step=1, unroll=4,
                        carry=jnp.zeros((L,), jnp.float32))
    def cnt(i, cnt):
        zp = lax.shift_right_logical(i, 6)
        row = i & 63
        for u in range(4):
            sl = pl.ds(u * L, L)
            mf = jnp.where(sd[zp, row, sl] < 0.0,
                           jnp.float32(1.0), jnp.float32(0.0))
            sd[zp, row, sl] = mf
            cnt = cnt + mf
        return cnt

    @plsc.parallel_loop(0, 256, step=1, unroll=3)
    def _(i):
        zp = lax.shift_right_logical(i, 6)
        row = i & 63
        for u in range(4):
            sl = pl.ds(u * L, L)
            process(dp[zp, row, sl], sd[zp, row, sl], histp)

    @plsc.parallel_loop(0, 256, step=1, unroll=3)
    def _(i):
        zp = lax.shift_right_logical(i, 6)
        row = i & 63
        for u in range(4):
            sl = pl.ds(u * L, L)
            process(dt[zp, row, sl], sd[zp, row, sl], histt)

    for chunk in range(8):
        accp = histp[pl.ds(chunk * L, L)]
        acct = histt[pl.ds(chunk * L, L)]
        for lane in range(1, L):
            accp = accp + histp[pl.ds(lane * HSTRIDE + chunk * L, L)]
            acct = acct + histt[pl.ds(lane * HSTRIDE + chunk * L, L)]
        orow[pl.ds(chunk * L, L)] = accp
        orow[pl.ds(128 + chunk * L, L)] = acct
    orow[pl.ds(256, L)] = cnt

    pltpu.sync_copy(orow, out_hbm.at[wid])


def _tc_body(x_ref, out_ref):
    x = x_ref[...]
    rows = lax.broadcasted_iota(jnp.int32, x.shape, 0)
    cols = lax.broadcasted_iota(jnp.int32, x.shape, 1)

    ii = lax.broadcasted_iota(jnp.int32, (128, 128), 0)
    jj = lax.broadcasted_iota(jnp.int32, (128, 128), 1)
    tri = jnp.where(ii >= jj, jnp.float32(1.0), jnp.float32(0.0))

    centers = (lax.broadcasted_iota(jnp.int32, (1, 128), 1).astype(jnp.float32)
               + 0.5) * BW

    cntsel = (cols >= 256) & (cols < 272)
    d95 = []
    ns = []
    for b in range(2):
        rowsel = (rows >= NS * b) & (rows < NS * (b + 1))
        srow = jnp.sum(jnp.where(rowsel, x, 0.0), axis=0, keepdims=True)
        n = jnp.sum(jnp.where(rowsel & cntsel, x, 0.0))
        ns.append(n)
        for dose in range(2):
            hg = srow[:, 128 * dose:128 * (dose + 1)]
            suffix = lax.dot(hg, tri, precision=lax.Precision.HIGHEST)
            cdf = suffix / jnp.maximum(n, 1.0)
            wd = jnp.exp((cdf - 0.95) ** 2 * (-1.0 / (2.0 * 0.1 * 0.1)))
            val = jnp.sum(wd * centers) / (jnp.sum(wd) + 1e-8)
            d95.append(jnp.where(n >= 100.0, val, 0.0))

    deficit = (jnp.maximum(d95[1] - d95[0], 0.0)
               + jnp.maximum(d95[3] - d95[2], 0.0))
    loss = 10.0 * deficit * 0.5
    total = jnp.where(ns[0] + ns[1] > 0.0, loss, 0.0)
    out_ref[...] = jnp.full((1, 1), total, jnp.float32)


@jax.jit
def kernel(pred, target, condition):
    mesh = plsc.VectorSubcoreMesh(core_axis_name="c", subcore_axis_name="s")
    partials = pl.kernel(
        _sc_body,
        out_type=jax.ShapeDtypeStruct((NW, OUTW), jnp.float32),
        mesh=mesh,
        compiler_params=pltpu.CompilerParams(needs_layout_passes=False),
        scratch_types=[
            pltpu.VMEM((4, 64, 64), jnp.float32),
            pltpu.VMEM((4, 64, 64), jnp.float32),
            pltpu.VMEM((4, 64, 64), jnp.float32),
            pltpu.VMEM((HWORDS,), jnp.float32),
            pltpu.VMEM((HWORDS,), jnp.float32),
            pltpu.VMEM((OUTW,), jnp.float32),
            pltpu.SemaphoreType.DMA,
            pltpu.SemaphoreType.DMA,
            pltpu.SemaphoreType.DMA,
        ],
    )(pred, target, condition)

    out = pl.pallas_call(
        _tc_body,
        out_shape=jax.ShapeDtypeStruct((1, 1), jnp.float32),
    )(partials)
    return out[0, 0]


# speedup vs baseline: 1.2568x; 1.2568x over previous
import functools
import math

import jax
import jax.numpy as jnp
from jax import lax
from jax.experimental import pallas as pl
from jax.experimental.pallas import tpu as pltpu
from jax.experimental.pallas import tpu_sc as plsc

N_BINS = 100
BW = 1.5 / N_BINS
INV_BW = N_BINS / 1.5
C1 = math.exp(-2.0)
C2 = math.exp(-8.0)

EV = [1.0000000005495513, 0.4999999725452058, 0.04166688603182874,
      0.001388275939515883, 2.5499197512124858e-05]
OD = [1.0000000002742597, 0.1666666611856373, 0.008333363992309727,
      0.00019834275296320354, 2.8254136118362913e-06]
PB = [0.9999999998500073, -0.4999999852294344, 0.12499976213798546,
      -0.020831893152229372, 0.0025999953457805794, -0.0002541579152189521,
      1.693882112112576e-05]

V = 64 * 64 * 64
NC = 2
NS = 16
L = 16
NW = NC * NS
VPW = V // NS
UNROLL = 4
ITERS = VPW // (L * UNROLL)
HSTRIDE = 129
HWORDS = L * HSTRIDE
OUTW = 384


def _sc_body(p_hbm, t_hbm, c_hbm, out_hbm, dp, dt, sd, histp, histt, orow,
             semp, semt, sems):
    c = lax.axis_index("c")
    s = lax.axis_index("s")
    wid = c * NS + s
    zoff = s * 4

    cp_p = pltpu.async_copy(p_hbm.at[c, 0, pl.ds(zoff, 4)], dp, semp)
    cp_t = pltpu.async_copy(t_hbm.at[c, 0, pl.ds(zoff, 4)], dt, semt)
    cp_s = pltpu.async_copy(c_hbm.at[c, 1, pl.ds(zoff, 4)], sd, sems)

    zeros = jnp.zeros((L,), jnp.float32)

    def zero_body(j, _):
        histp[pl.ds(j * L, L)] = zeros
        histt[pl.ds(j * L, L)] = zeros
        return 0

    lax.fori_loop(0, HSTRIDE, zero_body, 0)

    def zero_orow(j, _):
        orow[pl.ds(j * L, L)] = zeros
        return 0

    lax.fori_loop(0, OUTW // L, zero_orow, 0)

    cp_p.wait()
    cp_t.wait()
    cp_s.wait()

    lane_base = lax.iota(jnp.int32, L) * HSTRIDE

    def process(d, mf, href):
        t = d * INV_BW
        k = t.astype(jnp.int32)
        k = jnp.minimum(k, 96)
        f1 = (t - k.astype(jnp.float32)) * 2.0 - 1.0
        base = jnp.exp(-0.5 * (f1 * f1))
        g = jnp.exp(f1 + f1)
        gi = 1.0 / g
        w1 = base * (g * C1)
        w2 = base * ((g * g) * C2)
        v1 = base * (gi * C1)
        v2 = base * ((gi * gi) * C2)
        m1 = k >= 1
        m2 = k >= 2
        z = jnp.float32(0.0)
        ssum = ((base + w1) + w2) + (jnp.where(m1, v1, z) + jnp.where(m2, v2, z))
        scale = mf / (ssum + 1e-8)
        bidx = lane_base + k
        plsc.addupdate_scatter(href, [bidx], base * scale)
        plsc.addupdate_scatter(href, [bidx + 1], w1 * scale)
        plsc.addupdate_scatter(href, [bidx + 2], w2 * scale)
        plsc.addupdate_scatter(href, [bidx - 1], v1 * scale, mask=m1)
        plsc.addupdate_scatter(href, [bidx - 2], v2 * scale, mask=m2)

    @plsc.parallel_loop(0, 256, step=1, unroll=2,
                        carry=jnp.zeros((L,), jnp.float32))
    def cnt(i, cnt):
        zp = lax.shift_right_logical(i, 6)
        row = i & 63
        for u in range(4):
            sl = pl.ds(u * L, L)
            mf = jnp.where(sd[zp, row, sl] < 0.0,
                           jnp.float32(1.0), jnp.float32(0.0))
            sd[zp, row, sl] = mf
            process(dp[zp, row, sl], mf, histp)
            cnt = cnt + mf
        return cnt

    @plsc.parallel_loop(0, 256, step=1, unroll=2)
    def _(i):
        zp = lax.shift_right_logical(i, 6)
        row = i & 63
        for u in range(4):
            sl = pl.ds(u * L, L)
            process(dt[zp, row, sl], sd[zp, row, sl], histt)

    for chunk in range(8):
        accp = histp[pl.ds(chunk * L, L)]
        acct = histt[pl.ds(chunk * L, L)]
        for lane in range(1, L):
            accp = accp + histp[pl.ds(lane * HSTRIDE + chunk * L, L)]
            acct = acct + histt[pl.ds(lane * HSTRIDE + chunk * L, L)]
        orow[pl.ds(chunk * L, L)] = accp
        orow[pl.ds(128 + chunk * L, L)] = acct
    orow[pl.ds(256, L)] = cnt

    pltpu.sync_copy(orow, out_hbm.at[wid])


def _tc_body(x_ref, out_ref):
    x = x_ref[...]
    rows = lax.broadcasted_iota(jnp.int32, x.shape, 0)
    cols = lax.broadcasted_iota(jnp.int32, x.shape, 1)

    ii = lax.broadcasted_iota(jnp.int32, (128, 128), 0)
    jj = lax.broadcasted_iota(jnp.int32, (128, 128), 1)
    tri = jnp.where(ii >= jj, jnp.float32(1.0), jnp.float32(0.0))

    centers = (lax.broadcasted_iota(jnp.int32, (1, 128), 1).astype(jnp.float32)
               + 0.5) * BW

    cntsel = (cols >= 256) & (cols < 272)
    d95 = []
    ns = []
    for b in range(2):
        rowsel = (rows >= NS * b) & (rows < NS * (b + 1))
        srow = jnp.sum(jnp.where(rowsel, x, 0.0), axis=0, keepdims=True)
        n = jnp.sum(jnp.where(rowsel & cntsel, x, 0.0))
        ns.append(n)
        for dose in range(2):
            hg = srow[:, 128 * dose:128 * (dose + 1)]
            suffix = lax.dot(hg, tri, precision=lax.Precision.HIGHEST)
            cdf = suffix / jnp.maximum(n, 1.0)
            wd = jnp.exp((cdf - 0.95) ** 2 * (-1.0 / (2.0 * 0.1 * 0.1)))
            val = jnp.sum(wd * centers) / (jnp.sum(wd) + 1e-8)
            d95.append(jnp.where(n >= 100.0, val, 0.0))

    deficit = (jnp.maximum(d95[1] - d95[0], 0.0)
               + jnp.maximum(d95[3] - d95[2], 0.0))
    loss = 10.0 * deficit * 0.5
    total = jnp.where(ns[0] + ns[1] > 0.0, loss, 0.0)
    out_ref[...] = jnp.full((1, 1), total, jnp.float32)


@jax.jit
def kernel(pred, target, condition):
    mesh = plsc.VectorSubcoreMesh(core_axis_name="c", subcore_axis_name="s")
    partials = pl.kernel(
        _sc_body,
        out_type=jax.ShapeDtypeStruct((NW, OUTW), jnp.float32),
        mesh=mesh,
        compiler_params=pltpu.CompilerParams(needs_layout_passes=False),
        scratch_types=[
            pltpu.VMEM((4, 64, 64), jnp.float32),
            pltpu.VMEM((4, 64, 64), jnp.float32),
            pltpu.VMEM((4, 64, 64), jnp.float32),
            pltpu.VMEM((HWORDS,), jnp.float32),
            pltpu.VMEM((HWORDS,), jnp.float32),
            pltpu.VMEM((OUTW,), jnp.float32),
            pltpu.SemaphoreType.DMA,
            pltpu.SemaphoreType.DMA,
            pltpu.SemaphoreType.DMA,
        ],
    )(pred, target, condition)

    out = pl.pallas_call(
        _tc_body,
        out_shape=jax.ShapeDtypeStruct((1, 1), jnp.float32),
    )(partials)
    return out[0, 0]
